# baseline (speedup 1.0000x reference)
import functools

import jax
import jax.numpy as jnp
from jax import lax
from jax.experimental import pallas as pl
from jax.experimental.pallas import tpu as pltpu
from jax.experimental.pallas import tpu_sc as plsc

_B = 4096
_C = 1000
_K = 10

_NC = 2
_NS = 16
_L = 16
_NW = _NC * _NS
_RPW = _B // _NW
_NCH = _RPW // _L


def _sc_body(outT_hbm, labT_hbm, part_hbm, lab_v, row_v, acc_v, sem):
    cid = lax.axis_index("c")
    sid = lax.axis_index("s")
    wid = sid * _NC + cid
    base_row = wid * _RPW

    cp = pltpu.async_copy(outT_hbm.at[:, pl.ds(base_row, _RPW)], row_v, sem)
    pltpu.sync_copy(labT_hbm.at[:, pl.ds(base_row, _RPW)], lab_v)
    cp.wait()

    lane = lax.iota(jnp.int32, _L)

    def chunk(c, acc):
        col = c * _L + lane
        labs = [lab_v[j, pl.ds(c * _L, _L)] for j in range(_K)]
        vals = [plsc.load_gather(row_v, [labs[j], col]) for j in range(_K)]
        acc = acc + jnp.exp(vals[0])
        for j in range(1, _K):
            first = labs[j] != labs[0]
            for k in range(1, j):
                first = jnp.logical_and(first, labs[j] != labs[k])
            acc = acc + jnp.where(first, jnp.exp(vals[j]), 0.0)
        return acc

    acc_v[...] = lax.fori_loop(0, _NCH, chunk, jnp.zeros((_L,), jnp.float32))
    pltpu.sync_copy(acc_v, part_hbm.at[wid])


_sc_loss = functools.partial(
    pl.kernel,
    mesh=plsc.VectorSubcoreMesh(core_axis_name="c", subcore_axis_name="s",
                                num_cores=_NC, num_subcores=_NS),
    compiler_params=pltpu.CompilerParams(needs_layout_passes=False,
                                         use_tc_tiling_on_sc=True,
                                         skip_device_barrier=True),
    out_type=jax.ShapeDtypeStruct((_NW, _L), jnp.float32),
    scratch_types=[
        pltpu.VMEM((_K, _RPW), jnp.int32),
        pltpu.VMEM((_C, _RPW), jnp.float32),
        pltpu.VMEM((_L,), jnp.float32),
        pltpu.SemaphoreType.DMA,
    ],
)(_sc_body)


def kernel(outputs, complementary_labels):
    partials = _sc_loss(outputs.T, complementary_labels.T)
    return partials.sum() / outputs.shape[0]


# speedup vs baseline: 1.1093x; 1.1093x over previous
import functools

import jax
import jax.numpy as jnp
from jax import lax
from jax.experimental import pallas as pl
from jax.experimental.pallas import tpu as pltpu
from jax.experimental.pallas import tpu_sc as plsc

_B = 4096
_C = 1000
_K = 10

_NC = 2
_NS = 16
_L = 16
_NW = _NC * _NS
_RPW = _B // _NW
_NCH = _RPW // _L


def _sc_body(flat_hbm, labT_hbm, part_hbm, lab_v, idx_v, val_v, acc_v, sem):
    cid = lax.axis_index("c")
    sid = lax.axis_index("s")
    wid = sid * _NC + cid
    base_row = wid * _RPW

    pltpu.sync_copy(labT_hbm.at[:, pl.ds(base_row, _RPW)], lab_v)

    lane = lax.iota(jnp.int32, _L)
    copies = []
    for j in range(_K):
        for c in range(_NCH):
            lab = lab_v[j, pl.ds(c * _L, _L)]
            idx_v[j, pl.ds(c * _L, _L)] = (
                (lab >> 3) * 32768 + (lab & 7) * 128
                + (wid * 1024 + c * _L) + lane
            )
        copies.append(
            pltpu.async_copy(flat_hbm.at[idx_v.at[j]], val_v.at[j], sem))
    for cp in copies:
        cp.wait()

    def chunk(c, acc):
        labs = [lab_v[j, pl.ds(c * _L, _L)] for j in range(_K)]
        vals = [val_v[j, pl.ds(c * _L, _L)] for j in range(_K)]
        acc = acc + jnp.exp(vals[0])
        for j in range(1, _K):
            first = labs[j] != labs[0]
            for k in range(1, j):
                first = jnp.logical_and(first, labs[j] != labs[k])
            acc = acc + jnp.where(first, jnp.exp(vals[j]), 0.0)
        return acc

    acc_v[...] = lax.fori_loop(0, _NCH, chunk, jnp.zeros((_L,), jnp.float32))
    pltpu.sync_copy(acc_v, part_hbm.at[wid])


_sc_loss = functools.partial(
    pl.kernel,
    mesh=plsc.VectorSubcoreMesh(core_axis_name="c", subcore_axis_name="s",
                                num_cores=_NC, num_subcores=_NS),
    compiler_params=pltpu.CompilerParams(needs_layout_passes=False,
                                         use_tc_tiling_on_sc=True,
                                         skip_device_barrier=True),
    out_type=jax.ShapeDtypeStruct((_NW, _L), jnp.float32),
    scratch_types=[
        pltpu.VMEM((_K, _RPW), jnp.int32),
        pltpu.VMEM((_K, _RPW), jnp.int32),
        pltpu.VMEM((_K, _RPW), jnp.float32),
        pltpu.VMEM((_L,), jnp.float32),
        pltpu.SemaphoreType.DMA,
    ],
)(_sc_body)


def kernel(outputs, complementary_labels):
    flat = (outputs.T.reshape(_C // 8, 8, _B // 128, 128)
            .transpose(0, 2, 1, 3).reshape(-1))
    partials = _sc_loss(flat, complementary_labels.T)
    return partials.sum() / outputs.shape[0]
